# baseline (speedup 1.0000x reference)
import functools
import math

import jax
import jax.numpy as jnp
from jax.experimental import pallas as pl
from jax.experimental.pallas import tpu as pltpu

_NH = 12


def _fused_kernel(x_ref, wq_ref, wk_ref, wv_ref, wo_ref, bq_ref, bk_ref,
                  bv_ref, bo_ref, g_ref, be_ref, o_ref, *, nb, sb, dh, scale,
                  eps):
    x = x_ref[...]
    xb = x.astype(jnp.bfloat16)
    dn = (((1,), (1,)), ((), ()))

    q = jax.lax.dot_general(xb, wq_ref[...], dn,
                            preferred_element_type=jnp.float32) + bq_ref[...]
    k = jax.lax.dot_general(xb, wk_ref[...], dn,
                            preferred_element_type=jnp.float32) + bk_ref[...]
    v = jax.lax.dot_general(xb, wv_ref[...], dn,
                            preferred_element_type=jnp.float32) + bv_ref[...]

    qb = (q * scale).astype(jnp.bfloat16)
    kb = k.astype(jnp.bfloat16)
    vb = v.astype(jnp.bfloat16)

    ones_dh = jnp.ones((sb, dh), dtype=jnp.bfloat16)
    row_blocks = []
    for b in range(nb):
        rows = slice(b * sb, (b + 1) * sb)
        head_parts = []
        for h in range(_NH):
            cols = slice(h * dh, (h + 1) * dh)
            qh = qb[rows, cols]
            kh = kb[rows, cols]
            s = jax.lax.dot_general(qh, kh, dn,
                                    preferred_element_type=jnp.float32)
            s = s - jnp.max(s, axis=-1, keepdims=True)
            pb = jnp.exp2(s).astype(jnp.bfloat16)
            va = jnp.concatenate([vb[rows, cols], ones_dh], axis=1)
            nd = jnp.dot(pb, va, preferred_element_type=jnp.float32)
            head_parts.append((nd[:, :dh] / nd[:, dh:]).astype(jnp.bfloat16))
        row_blocks.append(jnp.concatenate(head_parts, axis=1))
    ctxb = jnp.concatenate(row_blocks, axis=0)

    h_out = jax.lax.dot_general(ctxb, wo_ref[...], dn,
                                preferred_element_type=jnp.float32)
    h_out = h_out + bo_ref[...] + x
    mean = jnp.mean(h_out, axis=-1, keepdims=True)
    c = h_out - mean
    var = jnp.mean(c * c, axis=-1, keepdims=True)
    y = c * jax.lax.rsqrt(var + eps) * g_ref[...] + be_ref[...]
    o_ref[...] = y.astype(o_ref.dtype)


def kernel(hidden_states, wq, wk, wv, wo, bq, bk, bv, bo, gamma, beta):
    B, S, H = hidden_states.shape
    nh = _NH
    dh = H // nh
    M = B * S
    dtype = hidden_states.dtype

    nb = 8
    while B % nb:
        nb -= 1
    tm = nb * S
    grid = (B // nb,)

    x2 = hidden_states.reshape(M, H)
    wqb = wq.astype(jnp.bfloat16)
    wkb = wk.astype(jnp.bfloat16)
    wvb = wv.astype(jnp.bfloat16)
    wob = wo.astype(jnp.bfloat16)
    bq2 = bq.reshape(1, H).astype(jnp.float32)
    bk2 = bk.reshape(1, H).astype(jnp.float32)
    bv2 = bv.reshape(1, H).astype(jnp.float32)
    bo2 = bo.reshape(1, H).astype(jnp.float32)
    g2 = gamma.reshape(1, H).astype(jnp.float32)
    be2 = beta.reshape(1, H).astype(jnp.float32)

    row_spec = pl.BlockSpec((tm, H), lambda i: (i, 0))
    wt_spec = pl.BlockSpec((H, H), lambda i: (0, 0))
    vec_spec = pl.BlockSpec((1, H), lambda i: (0, 0))

    out = pl.pallas_call(
        functools.partial(_fused_kernel, nb=nb, sb=S, dh=dh,
                          scale=math.log2(math.e) / math.sqrt(dh), eps=1e-12),
        out_shape=jax.ShapeDtypeStruct((M, H), dtype),
        grid=grid,
        in_specs=[row_spec, wt_spec, wt_spec, wt_spec, wt_spec,
                  vec_spec, vec_spec, vec_spec, vec_spec, vec_spec, vec_spec],
        out_specs=row_spec,
        compiler_params=pltpu.CompilerParams(
            dimension_semantics=("parallel",),
            vmem_limit_bytes=48 * 1024 * 1024,
        ),
    )(x2, wqb, wkb, wvb, wob, bq2, bk2, bv2, bo2, g2, be2)

    return out.reshape(B, S, H)


# speedup vs baseline: 2.0768x; 2.0768x over previous
import functools
import math

import jax
import jax.numpy as jnp
from jax.experimental import pallas as pl
from jax.experimental.pallas import tpu as pltpu

_NH = 12


def _fused_kernel(x_ref, wq_ref, wk_ref, wv_ref, wo_ref, bq_ref, bk_ref,
                  bv_ref, bo_ref, g_ref, be_ref, o_ref, *, nb, sb, dh, scale,
                  eps):
    x = x_ref[...]
    xb = x.astype(jnp.bfloat16)
    dn = (((1,), (1,)), ((), ()))

    q = jax.lax.dot_general(xb, wq_ref[...], dn,
                            preferred_element_type=jnp.float32) + bq_ref[...]
    k = jax.lax.dot_general(xb, wk_ref[...], dn,
                            preferred_element_type=jnp.float32) + bk_ref[...]
    v = jax.lax.dot_general(xb, wv_ref[...], dn,
                            preferred_element_type=jnp.float32) + bv_ref[...]

    qb = (q * scale).astype(jnp.bfloat16)
    kb = k.astype(jnp.bfloat16)
    vb = v.astype(jnp.bfloat16)

    ones_dh = jnp.ones((sb, dh), dtype=jnp.bfloat16)
    row_blocks = []
    for b in range(nb):
        rows = slice(b * sb, (b + 1) * sb)
        head_parts = []
        for h in range(_NH):
            cols = slice(h * dh, (h + 1) * dh)
            qh = qb[rows, cols]
            kh = kb[rows, cols]
            s = jax.lax.dot_general(qh, kh, dn,
                                    preferred_element_type=jnp.float32)
            s = s - jnp.max(s, axis=-1, keepdims=True)
            pb = jnp.exp2(s).astype(jnp.bfloat16)
            num = jnp.dot(pb, vb[rows, cols],
                          preferred_element_type=jnp.float32)
            den = jnp.dot(pb, ones_dh,
                          preferred_element_type=jnp.float32)
            head_parts.append((num / den).astype(jnp.bfloat16))
        row_blocks.append(jnp.concatenate(head_parts, axis=1))
    ctxb = jnp.concatenate(row_blocks, axis=0)

    h_out = jax.lax.dot_general(ctxb, wo_ref[...], dn,
                                preferred_element_type=jnp.float32)
    h_out = h_out + bo_ref[...] + x
    mean = jnp.mean(h_out, axis=-1, keepdims=True)
    c = h_out - mean
    var = jnp.mean(c * c, axis=-1, keepdims=True)
    y = c * jax.lax.rsqrt(var + eps) * g_ref[...] + be_ref[...]
    o_ref[...] = y.astype(o_ref.dtype)


def kernel(hidden_states, wq, wk, wv, wo, bq, bk, bv, bo, gamma, beta):
    B, S, H = hidden_states.shape
    nh = _NH
    dh = H // nh
    M = B * S
    dtype = hidden_states.dtype

    nb = 8
    while B % nb:
        nb -= 1
    tm = nb * S
    grid = (B // nb,)

    x2 = hidden_states.reshape(M, H)
    wqb = wq.astype(jnp.bfloat16)
    wkb = wk.astype(jnp.bfloat16)
    wvb = wv.astype(jnp.bfloat16)
    wob = wo.astype(jnp.bfloat16)
    bq2 = bq.reshape(1, H).astype(jnp.float32)
    bk2 = bk.reshape(1, H).astype(jnp.float32)
    bv2 = bv.reshape(1, H).astype(jnp.float32)
    bo2 = bo.reshape(1, H).astype(jnp.float32)
    g2 = gamma.reshape(1, H).astype(jnp.float32)
    be2 = beta.reshape(1, H).astype(jnp.float32)

    row_spec = pl.BlockSpec((tm, H), lambda i: (i, 0))
    wt_spec = pl.BlockSpec((H, H), lambda i: (0, 0))
    vec_spec = pl.BlockSpec((1, H), lambda i: (0, 0))

    out = pl.pallas_call(
        functools.partial(_fused_kernel, nb=nb, sb=S, dh=dh,
                          scale=math.log2(math.e) / math.sqrt(dh), eps=1e-12),
        out_shape=jax.ShapeDtypeStruct((M, H), dtype),
        grid=grid,
        in_specs=[row_spec, wt_spec, wt_spec, wt_spec, wt_spec,
                  vec_spec, vec_spec, vec_spec, vec_spec, vec_spec, vec_spec],
        out_specs=row_spec,
        compiler_params=pltpu.CompilerParams(
            dimension_semantics=("parallel",),
            vmem_limit_bytes=48 * 1024 * 1024,
        ),
    )(x2, wqb, wkb, wvb, wob, bq2, bk2, bv2, bo2, g2, be2)

    return out.reshape(B, S, H)


# speedup vs baseline: 2.0933x; 1.0079x over previous
import functools
import math

import jax
import jax.numpy as jnp
from jax.experimental import pallas as pl
from jax.experimental.pallas import tpu as pltpu

_NH = 12


def _fused_kernel(x_ref, wqkv_ref, wo_ref, bqkv_ref, bo_ref, g_ref,
                  be_ref, o_ref, *, nb, sb, dh, eps):
    x = x_ref[...]
    xb = x.astype(jnp.bfloat16)
    dn = (((1,), (1,)), ((), ()))
    H = x.shape[1]

    qkv = jax.lax.dot_general(xb, wqkv_ref[...], dn,
                              preferred_element_type=jnp.float32)
    qkvb = (qkv + bqkv_ref[...]).astype(jnp.bfloat16)
    qb = qkvb[:, :H]
    kb = qkvb[:, H:2 * H]
    vb = qkvb[:, 2 * H:]

    ones_dh = jnp.ones((sb, dh), dtype=jnp.bfloat16)
    row_blocks = []
    for b in range(nb):
        rows = slice(b * sb, (b + 1) * sb)
        head_parts = []
        for h in range(_NH):
            cols = slice(h * dh, (h + 1) * dh)
            qh = qb[rows, cols]
            kh = kb[rows, cols]
            s = jax.lax.dot_general(qh, kh, dn,
                                    preferred_element_type=jnp.float32)
            s = s - jnp.max(s, axis=-1, keepdims=True)
            pb = jnp.exp2(s).astype(jnp.bfloat16)
            num = jnp.dot(pb, vb[rows, cols],
                          preferred_element_type=jnp.float32)
            den = jnp.dot(pb, ones_dh,
                          preferred_element_type=jnp.float32)
            head_parts.append((num / den).astype(jnp.bfloat16))
        row_blocks.append(jnp.concatenate(head_parts, axis=1))
    ctxb = jnp.concatenate(row_blocks, axis=0)

    h_out = jax.lax.dot_general(ctxb, wo_ref[...], dn,
                                preferred_element_type=jnp.float32)
    h_out = h_out + bo_ref[...] + x
    mean = jnp.mean(h_out, axis=-1, keepdims=True)
    c = h_out - mean
    var = jnp.mean(c * c, axis=-1, keepdims=True)
    y = c * jax.lax.rsqrt(var + eps) * g_ref[...] + be_ref[...]
    o_ref[...] = y.astype(o_ref.dtype)


def kernel(hidden_states, wq, wk, wv, wo, bq, bk, bv, bo, gamma, beta):
    B, S, H = hidden_states.shape
    nh = _NH
    dh = H // nh
    M = B * S
    dtype = hidden_states.dtype

    nb = 8
    while B % nb:
        nb -= 1
    tm = nb * S
    grid = (B // nb,)

    x2 = hidden_states.reshape(M, H)
    scale = math.log2(math.e) / math.sqrt(dh)
    wqkv = jnp.concatenate([wq * scale, wk, wv], axis=0).astype(jnp.bfloat16)
    bqkv = jnp.concatenate([bq * scale, bk, bv]).reshape(1, 3 * H)
    bqkv = bqkv.astype(jnp.float32)
    wob = wo.astype(jnp.bfloat16)
    bo2 = bo.reshape(1, H).astype(jnp.float32)
    g2 = gamma.reshape(1, H).astype(jnp.float32)
    be2 = beta.reshape(1, H).astype(jnp.float32)

    row_spec = pl.BlockSpec((tm, H), lambda i: (i, 0))
    wqkv_spec = pl.BlockSpec((3 * H, H), lambda i: (0, 0))
    wt_spec = pl.BlockSpec((H, H), lambda i: (0, 0))
    vecw_spec = pl.BlockSpec((1, 3 * H), lambda i: (0, 0))
    vec_spec = pl.BlockSpec((1, H), lambda i: (0, 0))

    out = pl.pallas_call(
        functools.partial(_fused_kernel, nb=nb, sb=S, dh=dh, eps=1e-12),
        out_shape=jax.ShapeDtypeStruct((M, H), dtype),
        grid=grid,
        in_specs=[row_spec, wqkv_spec, wt_spec,
                  vecw_spec, vec_spec, vec_spec, vec_spec],
        out_specs=row_spec,
        compiler_params=pltpu.CompilerParams(
            dimension_semantics=("parallel",),
            vmem_limit_bytes=48 * 1024 * 1024,
        ),
    )(x2, wqkv, wob, bqkv, bo2, g2, be2)

    return out.reshape(B, S, H)
